# baseline (speedup 1.0000x reference)
import jax
import jax.numpy as jnp
import numpy as np
from jax.experimental import pallas as pl

_ATOM_DIMS = [119, 4, 12, 12, 10, 6, 6, 2, 2]
_OFFSETS = jnp.array(np.concatenate([[0], np.cumsum(_ATOM_DIMS)[:-1]]).astype(np.int32))
_L = 3
_G = 256


def _gcn_conv(h, src, dst, ew, W, b, n):
    loop = jnp.arange(n, dtype=src.dtype)
    s = jnp.concatenate([src, loop])
    d = jnp.concatenate([dst, loop])
    w = jnp.concatenate([ew, jnp.ones((n,), dtype=ew.dtype)])
    deg = jax.ops.segment_sum(w, d, num_segments=n)
    dinv = jnp.where(deg > 0, jax.lax.rsqrt(deg), 0.0)
    norm = dinv[s] * w * dinv[d]
    hw = h @ W
    msg = hw[s] * norm[:, None]
    return jax.ops.segment_sum(msg, d, num_segments=n) + b


def _copy_kernel(x_ref, o_ref):
    o_ref[...] = x_ref[...]


def kernel(x, edge_index, edge_attr, batch, atom_table, conv_W, conv_b,
           fc_W1, fc_b1, fc_W2, fc_b2, fc_W3, fc_b3):
    n = x.shape[0]
    src = edge_index[0]
    dst = edge_index[1]
    h = atom_table[x + _OFFSETS[None, :]].sum(axis=1)
    for layer in range(_L):
        h0 = _gcn_conv(h, src, dst, edge_attr[:, 0], conv_W[layer, 0], conv_b[layer, 0], n)
        h1 = _gcn_conv(h, src, dst, edge_attr[:, 1], conv_W[layer, 1], conv_b[layer, 1], n)
        h2 = _gcn_conv(h, src, dst, edge_attr[:, 2], conv_W[layer, 2], conv_b[layer, 2], n)
        hn = h0 + h1 + h2
        if layer < _L - 1:
            hn = jax.nn.relu(hn)
        hn = hn + h
        h = hn
    ones = jnp.ones((n,), dtype=h.dtype)
    counts = jax.ops.segment_sum(ones, batch, num_segments=_G)
    sums = jax.ops.segment_sum(h, batch, num_segments=_G)
    h_graph = sums / jnp.maximum(counts, 1.0)[:, None]
    out = h_graph @ fc_W1 + fc_b1
    out = out @ fc_W2 + fc_b2
    out = out @ fc_W3 + fc_b3
    out = pl.pallas_call(
        _copy_kernel,
        out_shape=jax.ShapeDtypeStruct(out.shape, out.dtype),
    )(out)
    return out


# speedup vs baseline: 8.3605x; 8.3605x over previous
import functools

import jax
import jax.numpy as jnp
import numpy as np
from jax import lax
from jax.experimental import pallas as pl
from jax.experimental.pallas import tpu as pltpu
from jax.experimental.pallas import tpu_sc as plsc

_ATOM_DIMS = [119, 4, 12, 12, 10, 6, 6, 2, 2]
_OFFSETS_NP = np.concatenate([[0], np.cumsum(_ATOM_DIMS)[:-1]]).astype(np.int32)

_N = 50000
_NP = 51200
_E = 800000
_EP = 802816
_EPB = _EP // 128
_D = 64
_G = 256
_L = 3

_K = 4
_TILES = 16
_ROWS_PER_TILE = _NP // _TILES
_ZC = 640
_NZ = _ROWS_PER_TILE // _ZC
_MAIN_BLOCKS = _EPB // _TILES
_MAIN_GROUPS = _MAIN_BLOCKS // _K
_NORM_BLOCKS = _EPB // (2 * _TILES)
_NORM_GROUPS = _NORM_BLOCKS // _K

_BN = 512


def _newton_rsqrt(x):
    y = plsc.bitcast(x, jnp.int32)
    y = jnp.int32(0x5F3759DF) - (y >> 1)
    y = plsc.bitcast(y, jnp.float32)
    for _ in range(3):
        y = y * (1.5 - 0.5 * x * y * y)
    return y


def _prep_body(s2, d2, w3, dinvN, nrm3,
               work_np, tbuf, idx_s, idx_d, wbuf, nbuf, nbuf4f, sem,
               part_sp, dinv_sp):
    core = lax.axis_index("c")
    tile = lax.axis_index("s")
    iota16 = lax.iota(jnp.int32, 16)
    n0 = tile * _ROWS_PER_TILE
    z16 = jnp.zeros((16,), jnp.float32)

    def zbuf4(i, _):
        nbuf4f[pl.ds(i * 16, 16)] = z16
        return 0
    lax.fori_loop(0, _ZC * 4 // 16, zbuf4, 0)

    for cc in range(3):
        def zwork(i, _):
            work_np[pl.ds(i * 16, 16)] = z16
            return 0
        lax.fori_loop(0, _NP // 16, zwork, 0)

        def deg_group(g, _):
            b0 = tile * _MAIN_BLOCKS + g * _K
            pltpu.sync_copy(d2.at[pl.ds(b0, _K)], idx_d)
            pltpu.sync_copy(w3.at[cc, pl.ds(b0, _K)], wbuf)
            for j in range(_K):
                def lanes(q, _):
                    dv = idx_d[j, pl.ds(q * 16, 16)]
                    wv = wbuf[j, pl.ds(q * 16, 16)]
                    plsc.addupdate_scatter(work_np, [dv], wv)
                    return 0
                lax.fori_loop(0, 8, lanes, 0)
            return 0
        lax.fori_loop(0, _MAIN_GROUPS, deg_group, 0)

        pltpu.sync_copy(work_np, part_sp.at[pl.ds(tile * _NP, _NP)])
        plsc.subcore_barrier()

        def zslice(i, _):
            work_np[pl.ds(n0 + i * 16, 16)] = z16
            return 0
        lax.fori_loop(0, _ROWS_PER_TILE // 16, zslice, 0)

        def redtile(tt, _):
            pltpu.sync_copy(part_sp.at[pl.ds(tt * _NP + n0, _ROWS_PER_TILE)], tbuf)

            def addv(i, _):
                work_np[pl.ds(n0 + i * 16, 16)] = (
                    work_np[pl.ds(n0 + i * 16, 16)] + tbuf[pl.ds(i * 16, 16)])
                return 0
            lax.fori_loop(0, _ROWS_PER_TILE // 16, addv, 0)
            return 0
        lax.fori_loop(0, _TILES, redtile, 0)

        def newt(i, _):
            v = work_np[pl.ds(n0 + i * 16, 16)]
            work_np[pl.ds(n0 + i * 16, 16)] = _newton_rsqrt(1.0 + v)
            return 0
        lax.fori_loop(0, _ROWS_PER_TILE // 16, newt, 0)
        pltpu.sync_copy(work_np.at[pl.ds(n0, _ROWS_PER_TILE)],
                        dinv_sp.at[pl.ds(cc * _NP + n0, _ROWS_PER_TILE)])
        plsc.subcore_barrier()

    @pl.when(core == 0)
    def _():
        for z in range(_NZ):
            node0 = n0 + z * _ZC
            for cc in range(3):
                pltpu.sync_copy(dinv_sp.at[pl.ds(cc * _NP + node0, _ZC)],
                                tbuf.at[pl.ds(cc * _ZC, _ZC)])

            def packrows(p, _):
                ridx = (p * 16 + iota16) * 4
                for cc in range(3):
                    v16 = tbuf[pl.ds(cc * _ZC + p * 16, 16)]
                    plsc.store_scatter(nbuf4f, [ridx + cc], v16)
                return 0
            lax.fori_loop(0, _ZC // 16, packrows, 0)
            pltpu.sync_copy(nbuf4f, dinvN.at[pl.ds(node0 * 4, _ZC * 4)])

    gtile = core * _TILES + tile
    for cc in range(3):
        pltpu.sync_copy(dinv_sp.at[pl.ds(cc * _NP, _NP)], work_np)

        def norm_group(g, _):
            b0 = gtile * _NORM_BLOCKS + g * _K
            pltpu.sync_copy(s2.at[pl.ds(b0, _K)], idx_s)
            pltpu.sync_copy(d2.at[pl.ds(b0, _K)], idx_d)
            pltpu.sync_copy(w3.at[cc, pl.ds(b0, _K)], wbuf)
            for j in range(_K):
                def lanes(q, _):
                    sv = idx_s[j, pl.ds(q * 16, 16)]
                    dv = idx_d[j, pl.ds(q * 16, 16)]
                    wv = wbuf[j, pl.ds(q * 16, 16)]
                    a = plsc.load_gather(work_np, [sv])
                    b = plsc.load_gather(work_np, [dv])
                    nbuf[j, pl.ds(q * 16, 16)] = a * wv * b
                    return 0
                lax.fori_loop(0, 8, lanes, 0)
            pltpu.sync_copy(nbuf, nrm3.at[cc, pl.ds(b0, _K)])
            return 0
        lax.fori_loop(0, _NORM_GROUPS, norm_group, 0)


def _prep_call(s2, d2, w3):
    mesh = plsc.VectorSubcoreMesh(core_axis_name="c", subcore_axis_name="s")
    fn = pl.kernel(
        _prep_body,
        out_type=(
            jax.ShapeDtypeStruct((_NP * 4,), jnp.float32),
            jax.ShapeDtypeStruct((3, _EPB, 128), jnp.float32),
        ),
        mesh=mesh,
        scratch_types=[
            pltpu.VMEM((_NP,), jnp.float32),
            pltpu.VMEM((_ROWS_PER_TILE,), jnp.float32),
            pltpu.VMEM((_K, 128), jnp.int32),
            pltpu.VMEM((_K, 128), jnp.int32),
            pltpu.VMEM((_K, 128), jnp.float32),
            pltpu.VMEM((_K, 128), jnp.float32),
            pltpu.VMEM((_ZC * 4,), jnp.float32),
            pltpu.SemaphoreType.DMA,
            pltpu.VMEM_SHARED((_TILES * _NP,), jnp.float32),
            pltpu.VMEM_SHARED((3 * _NP,), jnp.float32),
        ],
        compiler_params=pltpu.CompilerParams(needs_layout_passes=False,
                                             use_tc_tiling_on_sc=False),
        name="gcn_prep_sc",
    )
    return fn(s2, d2, w3)


def _conv_body(s2, d2, nrm3, h4, agg,
               zbuf, idx_s, idx_d, nrm, rows, sem, acc):
    core = lax.axis_index("c")
    tile = lax.axis_index("s")

    def zrow(i, _):
        zbuf[i, pl.ds(0, 16)] = jnp.zeros((16,), jnp.float32)
        return 0
    lax.fori_loop(0, _ZC, zrow, 0)

    for c in range(3):
        for half in range(2):
            qq = core * 2 + half
            for z in range(_NZ):
                pltpu.sync_copy(
                    zbuf, acc.at[pl.ds(tile * _ROWS_PER_TILE + z * _ZC, _ZC)])
            plsc.subcore_barrier()

            def group(g, _):
                b0 = tile * _MAIN_BLOCKS + g * _K
                pltpu.sync_copy(s2.at[pl.ds(b0, _K)], idx_s)
                pltpu.sync_copy(d2.at[pl.ds(b0, _K)], idx_d)
                pltpu.sync_copy(nrm3.at[c, pl.ds(b0, _K)], nrm)
                descs = [
                    pltpu.async_copy(h4.at[qq].at[idx_s.at[j]],
                                     rows.at[pl.ds(j * 128, 128)], sem)
                    for j in range(_K)
                ]
                for dsc in descs:
                    dsc.wait()
                for j in range(_K):
                    def scale(q, _):
                        nv = nrm[j, pl.ds(q * 16, 16)]
                        for lane in range(16):
                            sc = nv[lane]
                            r = j * 128 + q * 16 + lane
                            rows[r, pl.ds(0, 16)] = rows[r, pl.ds(0, 16)] * sc
                        return 0
                    lax.fori_loop(0, 8, scale, 0)
                for j in range(_K):
                    pltpu.sync_copy(rows.at[pl.ds(j * 128, 128)],
                                    acc.at[idx_d.at[j]], add=True)
                return 0
            lax.fori_loop(0, _MAIN_GROUPS, group, 0)
            plsc.subcore_barrier()

            for z in range(_NZ):
                r0 = tile * _ROWS_PER_TILE + z * _ZC
                pltpu.sync_copy(acc.at[pl.ds(r0, _ZC)],
                                agg.at[c, qq, pl.ds(r0, _ZC)])
            plsc.subcore_barrier()


def _conv_call(s2, d2, nrm3, h4):
    mesh = plsc.VectorSubcoreMesh(core_axis_name="c", subcore_axis_name="s")
    fn = pl.kernel(
        _conv_body,
        out_type=jax.ShapeDtypeStruct((3, 4, _NP, 16), jnp.float32),
        mesh=mesh,
        scratch_types=[
            pltpu.VMEM((_ZC, 16), jnp.float32),
            pltpu.VMEM((_K, 128), jnp.int32),
            pltpu.VMEM((_K, 128), jnp.int32),
            pltpu.VMEM((_K, 128), jnp.float32),
            pltpu.VMEM((_K * 128, 16), jnp.float32),
            pltpu.SemaphoreType.DMA,
            pltpu.VMEM_SHARED((_NP, 16), jnp.float32),
        ],
        compiler_params=pltpu.CompilerParams(needs_layout_passes=False,
                                             use_tc_tiling_on_sc=False),
        name="gcn_conv_sc",
    )
    return fn(s2, d2, nrm3, h4)


def _encoder_body(x_ref, off_ref, tab_ref, o_ref):
    xb = x_ref[...]
    idx = xb + off_ref[...]
    cols = lax.broadcasted_iota(jnp.int32, (_BN, 256), 1)
    m = jnp.zeros((_BN, 256), jnp.float32)
    for k in range(9):
        m = m + (idx[:, k:k + 1] == cols).astype(jnp.float32)
    h = jnp.dot(m, tab_ref[...], preferred_element_type=jnp.float32)
    for q in range(4):
        o_ref[q, :, :] = h[:, q * 16:(q + 1) * 16]


def _encoder_call(x_pad, offs, tab_pad):
    return pl.pallas_call(
        _encoder_body,
        grid=(_NP // _BN,),
        in_specs=[
            pl.BlockSpec((_BN, 9), lambda i: (i, 0)),
            pl.BlockSpec((1, 9), lambda i: (0, 0)),
            pl.BlockSpec((256, 64), lambda i: (0, 0)),
        ],
        out_specs=pl.BlockSpec((4, _BN, 16), lambda i: (0, i, 0)),
        out_shape=jax.ShapeDtypeStruct((4, _NP, 16), jnp.float32),
    )(x_pad, offs, tab_pad)


def _update_body(relu, agg_ref, h4_ref, dv_ref, w_ref, b_ref, o_ref):
    dv = dv_ref[...]
    hq = [h4_ref[q] for q in range(4)]
    acc = jnp.zeros((_BN, 64), jnp.float32)
    for c in range(3):
        dc = dv[:, c:c + 1]
        idg = dc * dc
        for q in range(4):
            aq = agg_ref[c, q] + hq[q] * idg
            acc = acc + jnp.dot(aq, w_ref[c, q * 16:(q + 1) * 16, :],
                                preferred_element_type=jnp.float32)
    bsum = b_ref[0:1, :] + b_ref[1:2, :] + b_ref[2:3, :]
    acc = acc + bsum
    if relu:
        acc = jnp.maximum(acc, 0.0)
    for q in range(4):
        o_ref[q, :, :] = acc[:, q * 16:(q + 1) * 16] + hq[q]


def _update_call(agg, h4, dinvN, wl, bl, relu):
    return pl.pallas_call(
        functools.partial(_update_body, relu),
        grid=(_NP // _BN,),
        in_specs=[
            pl.BlockSpec((3, 4, _BN, 16), lambda i: (0, 0, i, 0)),
            pl.BlockSpec((4, _BN, 16), lambda i: (0, i, 0)),
            pl.BlockSpec((_BN, 4), lambda i: (i, 0)),
            pl.BlockSpec((3, 64, 64), lambda i: (0, 0, 0)),
            pl.BlockSpec((3, 64), lambda i: (0, 0)),
        ],
        out_specs=pl.BlockSpec((4, _BN, 16), lambda i: (0, i, 0)),
        out_shape=jax.ShapeDtypeStruct((4, _NP, 16), jnp.float32),
    )(agg, h4, dinvN, wl, bl)


def _pool_body(h2_ref, b_ref, w1_ref, b1_ref, w2_ref, b2_ref, w3_ref, b3_ref,
               o_ref, acc_ref):
    i = pl.program_id(0)

    @pl.when(i == 0)
    def _():
        acc_ref[...] = jnp.zeros((_G, 72), jnp.float32)

    ones = jnp.ones((_BN, 8), jnp.float32)
    hb = jnp.concatenate([h2_ref[0], h2_ref[1], h2_ref[2], h2_ref[3], ones],
                         axis=1)
    grp = lax.broadcasted_iota(jnp.int32, (_BN, _G), 1)
    onehot = (b_ref[...] == grp).astype(jnp.float32)
    acc_ref[...] = acc_ref[...] + lax.dot_general(
        onehot, hb, (((0,), (0,)), ((), ())), preferred_element_type=jnp.float32)

    @pl.when(i == _NP // _BN - 1)
    def _():
        a = acc_ref[...]
        hg = a[:, :64] / jnp.maximum(a[:, 64:65], 1.0)
        o = jnp.dot(hg, w1_ref[...], preferred_element_type=jnp.float32) + b1_ref[...]
        o = jnp.dot(o, w2_ref[...], preferred_element_type=jnp.float32) + b2_ref[...]
        o = jnp.dot(o, w3_ref[...], preferred_element_type=jnp.float32) + b3_ref[...]
        o_ref[...] = o


def _pool_call(h2, batch2, fc_W1, fc_b1, fc_W2, fc_b2, fc_W3, fc_b3):
    return pl.pallas_call(
        _pool_body,
        grid=(_NP // _BN,),
        in_specs=[
            pl.BlockSpec((4, _BN, 16), lambda i: (0, i, 0)),
            pl.BlockSpec((_BN, 1), lambda i: (i, 0)),
            pl.BlockSpec((64, 64), lambda i: (0, 0)),
            pl.BlockSpec((1, 64), lambda i: (0, 0)),
            pl.BlockSpec((64, 64), lambda i: (0, 0)),
            pl.BlockSpec((1, 64), lambda i: (0, 0)),
            pl.BlockSpec((64, 1), lambda i: (0, 0)),
            pl.BlockSpec((1, 1), lambda i: (0, 0)),
        ],
        out_specs=pl.BlockSpec((_G, 1), lambda i: (0, 0)),
        out_shape=jax.ShapeDtypeStruct((_G, 1), jnp.float32),
        scratch_shapes=[pltpu.VMEM((_G, 72), jnp.float32)],
    )(h2, batch2, fc_W1, fc_b1, fc_W2, fc_b2, fc_W3, fc_b3)


def kernel(x, edge_index, edge_attr, batch, atom_table, conv_W, conv_b,
           fc_W1, fc_b1, fc_W2, fc_b2, fc_W3, fc_b3):
    s = edge_index[0]
    d = edge_index[1]
    s2 = jnp.pad(s, (0, _EP - _E)).reshape(_EPB, 128)
    d2 = jnp.pad(d, (0, _EP - _E)).reshape(_EPB, 128)
    w3 = jnp.pad(edge_attr.T, ((0, 0), (0, _EP - _E))).reshape(3, _EPB, 128)
    x_pad = jnp.pad(x, ((0, _NP - _N), (0, 0)))
    batch2 = jnp.pad(batch, (0, _NP - _N), constant_values=_G).reshape(_NP, 1)
    tab_pad = jnp.pad(atom_table, ((0, 256 - atom_table.shape[0]), (0, 0)))
    offs = jnp.asarray(_OFFSETS_NP).reshape(1, 9)

    h2 = _encoder_call(x_pad, offs, tab_pad)
    dinvN, nrm3 = _prep_call(s2, d2, w3)
    dinvN = dinvN.reshape(_NP, 4)

    for layer in range(_L):
        agg = _conv_call(s2, d2, nrm3, h2)
        h2 = _update_call(agg, h2, dinvN, conv_W[layer], conv_b[layer],
                          relu=(layer < _L - 1))

    return _pool_call(h2, batch2, fc_W1, fc_b1.reshape(1, 64),
                      fc_W2, fc_b2.reshape(1, 64),
                      fc_W3.reshape(64, 1), fc_b3.reshape(1, 1))


# speedup vs baseline: 14.6248x; 1.7493x over previous
import functools

import jax
import jax.numpy as jnp
import numpy as np
from jax import lax
from jax.experimental import pallas as pl
from jax.experimental.pallas import tpu as pltpu
from jax.experimental.pallas import tpu_sc as plsc

_ATOM_DIMS = [119, 4, 12, 12, 10, 6, 6, 2, 2]
_OFFSETS_NP = np.concatenate([[0], np.cumsum(_ATOM_DIMS)[:-1]]).astype(np.int32)

_N = 50000
_NP = 51200
_E = 800000
_EP = 802816
_EPB = _EP // 128
_D = 64
_G = 256
_L = 3

_K = 8
_TILES = 16
_ROWS_PER_TILE = _NP // _TILES
_ZC = 640
_NZ = _ROWS_PER_TILE // _ZC
_MAIN_BLOCKS = _EPB // _TILES
_MAIN_GROUPS = _MAIN_BLOCKS // _K
_NORM_BLOCKS = _EPB // (2 * _TILES)
_NORM_GROUPS = _NORM_BLOCKS // _K

_BN = 512


def _newton_rsqrt(x):
    y = plsc.bitcast(x, jnp.int32)
    y = jnp.int32(0x5F3759DF) - (y >> 1)
    y = plsc.bitcast(y, jnp.float32)
    for _ in range(3):
        y = y * (1.5 - 0.5 * x * y * y)
    return y


def _prep_body(s2, d2, w3, dinvN, nrm3,
               work_np, tbuf, idx_s, idx_d, wbuf, nbuf, nbuf4f, sem,
               part_sp, dinv_sp):
    core = lax.axis_index("c")
    tile = lax.axis_index("s")
    iota16 = lax.iota(jnp.int32, 16)
    n0 = tile * _ROWS_PER_TILE
    z16 = jnp.zeros((16,), jnp.float32)

    def zbuf4(i, _):
        nbuf4f[pl.ds(i * 16, 16)] = z16
        return 0
    lax.fori_loop(0, _ZC * 4 // 16, zbuf4, 0)

    for cc in range(3):
        def zwork(i, _):
            work_np[pl.ds(i * 16, 16)] = z16
            return 0
        lax.fori_loop(0, _NP // 16, zwork, 0)

        def deg_group(g, _):
            b0 = tile * _MAIN_BLOCKS + g * _K
            pltpu.sync_copy(d2.at[pl.ds(b0, _K)], idx_d)
            pltpu.sync_copy(w3.at[cc, pl.ds(b0, _K)], wbuf)
            for j in range(_K):
                def lanes(q, _):
                    dv = idx_d[j, pl.ds(q * 16, 16)]
                    wv = wbuf[j, pl.ds(q * 16, 16)]
                    plsc.addupdate_scatter(work_np, [dv], wv)
                    return 0
                lax.fori_loop(0, 8, lanes, 0)
            return 0
        lax.fori_loop(0, _MAIN_GROUPS, deg_group, 0)

        pltpu.sync_copy(work_np, part_sp.at[pl.ds(tile * _NP, _NP)])
        plsc.subcore_barrier()

        def zslice(i, _):
            work_np[pl.ds(n0 + i * 16, 16)] = z16
            return 0
        lax.fori_loop(0, _ROWS_PER_TILE // 16, zslice, 0)

        def redtile(tt, _):
            pltpu.sync_copy(part_sp.at[pl.ds(tt * _NP + n0, _ROWS_PER_TILE)], tbuf)

            def addv(i, _):
                work_np[pl.ds(n0 + i * 16, 16)] = (
                    work_np[pl.ds(n0 + i * 16, 16)] + tbuf[pl.ds(i * 16, 16)])
                return 0
            lax.fori_loop(0, _ROWS_PER_TILE // 16, addv, 0)
            return 0
        lax.fori_loop(0, _TILES, redtile, 0)

        def newt(i, _):
            v = work_np[pl.ds(n0 + i * 16, 16)]
            work_np[pl.ds(n0 + i * 16, 16)] = _newton_rsqrt(1.0 + v)
            return 0
        lax.fori_loop(0, _ROWS_PER_TILE // 16, newt, 0)
        pltpu.sync_copy(work_np.at[pl.ds(n0, _ROWS_PER_TILE)],
                        dinv_sp.at[pl.ds(cc * _NP + n0, _ROWS_PER_TILE)])
        plsc.subcore_barrier()

    @pl.when(core == 0)
    def _():
        for z in range(_NZ):
            node0 = n0 + z * _ZC
            for cc in range(3):
                pltpu.sync_copy(dinv_sp.at[pl.ds(cc * _NP + node0, _ZC)],
                                tbuf.at[pl.ds(cc * _ZC, _ZC)])

            def packrows(p, _):
                ridx = (p * 16 + iota16) * 4
                for cc in range(3):
                    v16 = tbuf[pl.ds(cc * _ZC + p * 16, 16)]
                    plsc.store_scatter(nbuf4f, [ridx + cc], v16)
                return 0
            lax.fori_loop(0, _ZC // 16, packrows, 0)
            pltpu.sync_copy(nbuf4f, dinvN.at[pl.ds(node0 * 4, _ZC * 4)])

    gtile = core * _TILES + tile
    for cc in range(3):
        pltpu.sync_copy(dinv_sp.at[pl.ds(cc * _NP, _NP)], work_np)

        def norm_group(g, _):
            b0 = gtile * _NORM_BLOCKS + g * _K
            pltpu.sync_copy(s2.at[pl.ds(b0, _K)], idx_s)
            pltpu.sync_copy(d2.at[pl.ds(b0, _K)], idx_d)
            pltpu.sync_copy(w3.at[cc, pl.ds(b0, _K)], wbuf)
            for j in range(_K):
                def lanes(q, _):
                    sv = idx_s[j, pl.ds(q * 16, 16)]
                    dv = idx_d[j, pl.ds(q * 16, 16)]
                    wv = wbuf[j, pl.ds(q * 16, 16)]
                    a = plsc.load_gather(work_np, [sv])
                    b = plsc.load_gather(work_np, [dv])
                    nbuf[j, pl.ds(q * 16, 16)] = a * wv * b
                    return 0
                lax.fori_loop(0, 8, lanes, 0)
            pltpu.sync_copy(nbuf, nrm3.at[cc, pl.ds(b0, _K)])
            return 0
        lax.fori_loop(0, _NORM_GROUPS, norm_group, 0)


def _prep_call(s2, d2, w3):
    mesh = plsc.VectorSubcoreMesh(core_axis_name="c", subcore_axis_name="s")
    fn = pl.kernel(
        _prep_body,
        out_type=(
            jax.ShapeDtypeStruct((_NP * 4,), jnp.float32),
            jax.ShapeDtypeStruct((3, _EPB, 128), jnp.float32),
        ),
        mesh=mesh,
        scratch_types=[
            pltpu.VMEM((_NP,), jnp.float32),
            pltpu.VMEM((_ROWS_PER_TILE,), jnp.float32),
            pltpu.VMEM((_K, 128), jnp.int32),
            pltpu.VMEM((_K, 128), jnp.int32),
            pltpu.VMEM((_K, 128), jnp.float32),
            pltpu.VMEM((_K, 128), jnp.float32),
            pltpu.VMEM((_ZC * 4,), jnp.float32),
            pltpu.SemaphoreType.DMA,
            pltpu.VMEM_SHARED((_TILES * _NP,), jnp.float32),
            pltpu.VMEM_SHARED((3 * _NP,), jnp.float32),
        ],
        compiler_params=pltpu.CompilerParams(needs_layout_passes=False,
                                             use_tc_tiling_on_sc=False),
        name="gcn_prep_sc",
    )
    return fn(s2, d2, w3)


def _conv_body(s2, d2, nrm3, h4, agg,
               zbuf, idx_s, idx_d, nrm, rows, semA, semB, acc):
    core = lax.axis_index("c")
    tile = lax.axis_index("s")

    def zrow(i, _):
        zbuf[i, pl.ds(0, 16)] = jnp.zeros((16,), jnp.float32)
        return 0
    lax.fori_loop(0, _ZC, zrow, 0)

    nconv_groups = _MAIN_BLOCKS // _K

    def conv_pass(cp, _):
        c = cp >> 1
        half = cp & 1
        qq = core * 2 + half
        for z in range(_NZ):
            pltpu.sync_copy(
                zbuf, acc.at[pl.ds(tile * _ROWS_PER_TILE + z * _ZC, _ZC)])
        plsc.subcore_barrier()

        def load_grp(g, b):
            b0 = tile * _MAIN_BLOCKS + g * _K
            pltpu.sync_copy(s2.at[pl.ds(b0, _K)], idx_s.at[b])
            pltpu.sync_copy(d2.at[pl.ds(b0, _K)], idx_d.at[b])
            pltpu.sync_copy(nrm3.at[c, pl.ds(b0, _K)], nrm.at[b])

        def fire_gathers(b, sem):
            for j in range(_K):
                pltpu.async_copy(h4.at[qq].at[idx_s.at[b, j]],
                                 rows.at[b].at[pl.ds(j * 128, 128)], sem)

        def wait_gathers(b, sem):
            for j in range(_K):
                pltpu.make_async_copy(
                    h4.at[qq].at[idx_s.at[b, j]],
                    rows.at[b].at[pl.ds(j * 128, 128)], sem).wait()

        def scale_scatter(b, sem):
            for j in range(_K):
                def scale(q, _):
                    nv = nrm[b, j, pl.ds(q * 16, 16)]
                    for lane in range(16):
                        sc = nv[lane]
                        r = j * 128 + q * 16 + lane
                        rows[b, r, pl.ds(0, 16)] = rows[b, r, pl.ds(0, 16)] * sc
                    return 0
                lax.fori_loop(0, 8, scale, 0)
            for j in range(_K):
                pltpu.async_copy(rows.at[b].at[pl.ds(j * 128, 128)],
                                 acc.at[idx_d.at[b, j]], sem, add=True)
            for j in range(_K):
                pltpu.make_async_copy(rows.at[b].at[pl.ds(j * 128, 128)],
                                      acc.at[idx_d.at[b, j]], sem).wait()

        load_grp(0, 0)
        fire_gathers(0, semA)

        def pipe(i, _):
            load_grp(2 * i + 1, 1)
            fire_gathers(1, semB)
            wait_gathers(0, semA)
            scale_scatter(0, semA)
            load_grp(2 * i + 2, 0)
            fire_gathers(0, semA)
            wait_gathers(1, semB)
            scale_scatter(1, semB)
            return 0
        lax.fori_loop(0, (nconv_groups - 1) // 2, pipe, 0)

        wait_gathers(0, semA)
        scale_scatter(0, semA)
        plsc.subcore_barrier()

        for z in range(_NZ):
            r0 = tile * _ROWS_PER_TILE + z * _ZC
            pltpu.sync_copy(acc.at[pl.ds(r0, _ZC)],
                            agg.at[c, qq, pl.ds(r0, _ZC)])
        plsc.subcore_barrier()
        return 0

    lax.fori_loop(0, 6, conv_pass, 0)


def _conv_call(s2, d2, nrm3, h4):
    mesh = plsc.VectorSubcoreMesh(core_axis_name="c", subcore_axis_name="s")
    fn = pl.kernel(
        _conv_body,
        out_type=jax.ShapeDtypeStruct((3, 4, _NP, 16), jnp.float32),
        mesh=mesh,
        scratch_types=[
            pltpu.VMEM((_ZC, 16), jnp.float32),
            pltpu.VMEM((2, _K, 128), jnp.int32),
            pltpu.VMEM((2, _K, 128), jnp.int32),
            pltpu.VMEM((2, _K, 128), jnp.float32),
            pltpu.VMEM((2, _K * 128, 16), jnp.float32),
            pltpu.SemaphoreType.DMA,
            pltpu.SemaphoreType.DMA,
            pltpu.VMEM_SHARED((_NP, 16), jnp.float32),
        ],
        compiler_params=pltpu.CompilerParams(needs_layout_passes=False,
                                             use_tc_tiling_on_sc=False),
        name="gcn_conv_sc",
    )
    return fn(s2, d2, nrm3, h4)


def _encoder_body(x_ref, off_ref, tab_ref, o_ref):
    xb = x_ref[...]
    idx = xb + off_ref[...]
    cols = lax.broadcasted_iota(jnp.int32, (_BN, 256), 1)
    m = jnp.zeros((_BN, 256), jnp.float32)
    for k in range(9):
        m = m + (idx[:, k:k + 1] == cols).astype(jnp.float32)
    h = jnp.dot(m, tab_ref[...], preferred_element_type=jnp.float32)
    for q in range(4):
        o_ref[q, :, :] = h[:, q * 16:(q + 1) * 16]


def _encoder_call(x_pad, offs, tab_pad):
    return pl.pallas_call(
        _encoder_body,
        grid=(_NP // _BN,),
        in_specs=[
            pl.BlockSpec((_BN, 9), lambda i: (i, 0)),
            pl.BlockSpec((1, 9), lambda i: (0, 0)),
            pl.BlockSpec((256, 64), lambda i: (0, 0)),
        ],
        out_specs=pl.BlockSpec((4, _BN, 16), lambda i: (0, i, 0)),
        out_shape=jax.ShapeDtypeStruct((4, _NP, 16), jnp.float32),
    )(x_pad, offs, tab_pad)


def _update_body(relu, agg_ref, h4_ref, dv_ref, w_ref, b_ref, o_ref):
    dv = dv_ref[...]
    hq = [h4_ref[q] for q in range(4)]
    acc = jnp.zeros((_BN, 64), jnp.float32)
    for c in range(3):
        dc = dv[:, c:c + 1]
        idg = dc * dc
        for q in range(4):
            aq = agg_ref[c, q] + hq[q] * idg
            acc = acc + jnp.dot(aq, w_ref[c, q * 16:(q + 1) * 16, :],
                                preferred_element_type=jnp.float32)
    bsum = b_ref[0:1, :] + b_ref[1:2, :] + b_ref[2:3, :]
    acc = acc + bsum
    if relu:
        acc = jnp.maximum(acc, 0.0)
    for q in range(4):
        o_ref[q, :, :] = acc[:, q * 16:(q + 1) * 16] + hq[q]


def _update_call(agg, h4, dinvN, wl, bl, relu):
    return pl.pallas_call(
        functools.partial(_update_body, relu),
        grid=(_NP // _BN,),
        in_specs=[
            pl.BlockSpec((3, 4, _BN, 16), lambda i: (0, 0, i, 0)),
            pl.BlockSpec((4, _BN, 16), lambda i: (0, i, 0)),
            pl.BlockSpec((_BN, 4), lambda i: (i, 0)),
            pl.BlockSpec((3, 64, 64), lambda i: (0, 0, 0)),
            pl.BlockSpec((3, 64), lambda i: (0, 0)),
        ],
        out_specs=pl.BlockSpec((4, _BN, 16), lambda i: (0, i, 0)),
        out_shape=jax.ShapeDtypeStruct((4, _NP, 16), jnp.float32),
    )(agg, h4, dinvN, wl, bl)


def _pool_body(h2_ref, b_ref, w1_ref, b1_ref, w2_ref, b2_ref, w3_ref, b3_ref,
               o_ref, acc_ref):
    i = pl.program_id(0)

    @pl.when(i == 0)
    def _():
        acc_ref[...] = jnp.zeros((_G, 72), jnp.float32)

    ones = jnp.ones((_BN, 8), jnp.float32)
    hb = jnp.concatenate([h2_ref[0], h2_ref[1], h2_ref[2], h2_ref[3], ones],
                         axis=1)
    grp = lax.broadcasted_iota(jnp.int32, (_BN, _G), 1)
    onehot = (b_ref[...] == grp).astype(jnp.float32)
    acc_ref[...] = acc_ref[...] + lax.dot_general(
        onehot, hb, (((0,), (0,)), ((), ())), preferred_element_type=jnp.float32)

    @pl.when(i == _NP // _BN - 1)
    def _():
        a = acc_ref[...]
        hg = a[:, :64] / jnp.maximum(a[:, 64:65], 1.0)
        o = jnp.dot(hg, w1_ref[...], preferred_element_type=jnp.float32) + b1_ref[...]
        o = jnp.dot(o, w2_ref[...], preferred_element_type=jnp.float32) + b2_ref[...]
        o = jnp.dot(o, w3_ref[...], preferred_element_type=jnp.float32) + b3_ref[...]
        o_ref[...] = o


def _pool_call(h2, batch2, fc_W1, fc_b1, fc_W2, fc_b2, fc_W3, fc_b3):
    return pl.pallas_call(
        _pool_body,
        grid=(_NP // _BN,),
        in_specs=[
            pl.BlockSpec((4, _BN, 16), lambda i: (0, i, 0)),
            pl.BlockSpec((_BN, 1), lambda i: (i, 0)),
            pl.BlockSpec((64, 64), lambda i: (0, 0)),
            pl.BlockSpec((1, 64), lambda i: (0, 0)),
            pl.BlockSpec((64, 64), lambda i: (0, 0)),
            pl.BlockSpec((1, 64), lambda i: (0, 0)),
            pl.BlockSpec((64, 1), lambda i: (0, 0)),
            pl.BlockSpec((1, 1), lambda i: (0, 0)),
        ],
        out_specs=pl.BlockSpec((_G, 1), lambda i: (0, 0)),
        out_shape=jax.ShapeDtypeStruct((_G, 1), jnp.float32),
        scratch_shapes=[pltpu.VMEM((_G, 72), jnp.float32)],
    )(h2, batch2, fc_W1, fc_b1, fc_W2, fc_b2, fc_W3, fc_b3)


def kernel(x, edge_index, edge_attr, batch, atom_table, conv_W, conv_b,
           fc_W1, fc_b1, fc_W2, fc_b2, fc_W3, fc_b3):
    s = edge_index[0]
    d = edge_index[1]
    s2 = jnp.pad(s, (0, _EP - _E)).reshape(_EPB, 128)
    d2 = jnp.pad(d, (0, _EP - _E)).reshape(_EPB, 128)
    w3 = jnp.pad(edge_attr.T, ((0, 0), (0, _EP - _E))).reshape(3, _EPB, 128)
    x_pad = jnp.pad(x, ((0, _NP - _N), (0, 0)))
    batch2 = jnp.pad(batch, (0, _NP - _N), constant_values=_G).reshape(_NP, 1)
    tab_pad = jnp.pad(atom_table, ((0, 256 - atom_table.shape[0]), (0, 0)))
    offs = jnp.asarray(_OFFSETS_NP).reshape(1, 9)

    h2 = _encoder_call(x_pad, offs, tab_pad)
    dinvN, nrm3 = _prep_call(s2, d2, w3)
    dinvN = dinvN.reshape(_NP, 4)

    for layer in range(_L):
        agg = _conv_call(s2, d2, nrm3, h2)
        h2 = _update_call(agg, h2, dinvN, conv_W[layer], conv_b[layer],
                          relu=(layer < _L - 1))

    return _pool_call(h2, batch2, fc_W1, fc_b1.reshape(1, 64),
                      fc_W2, fc_b2.reshape(1, 64),
                      fc_W3.reshape(64, 1), fc_b3.reshape(1, 1))
